# baseline (speedup 1.0000x reference)
import functools

import jax
import jax.numpy as jnp
from jax import lax
from jax.experimental import pallas as pl
from jax.experimental.pallas import tpu as pltpu
from jax.experimental.pallas import tpu_sc as plsc

_NW = 32
_LANES = 16
_NBUF = 4


def _build_sc_gather(n_chunks_total, rows, rows_pad, d_model, boundary,
                     lo_off, hi_off):
    ch_per_w = n_chunks_total // _NW
    assert ch_per_w % _NBUF == 0
    n_groups = ch_per_w // _NBUF
    idx_per_w = ch_per_w * rows_pad
    mesh = plsc.VectorSubcoreMesh(core_axis_name="c", subcore_axis_name="s")

    @functools.partial(
        pl.kernel,
        mesh=mesh,
        out_type=jax.ShapeDtypeStruct(
            (n_chunks_total, rows, d_model // 128, 128), jnp.float32),
        scratch_types=[
            pltpu.VMEM((idx_per_w,), jnp.int32),
        ]
        + [pltpu.VMEM((rows, d_model // 128, 128), jnp.float32)
           for _ in range(_NBUF)]
        + [pltpu.SemaphoreType.DMA for _ in range(2 * _NBUF)],
    )
    def k(w_hbm, idx_hbm, out_hbm, idx_v, *bufsem):
        bufs = bufsem[:_NBUF]
        gsem = bufsem[_NBUF:2 * _NBUF]
        ssem = bufsem[2 * _NBUF:]
        wid = lax.axis_index("s") * 2 + lax.axis_index("c")
        base_c = wid * ch_per_w
        pltpu.sync_copy(idx_hbm.at[pl.ds(wid * idx_per_w, idx_per_w)], idx_v)

        off = jnp.where(wid < boundary, lo_off, hi_off).astype(jnp.int32)

        def add_off(i, carry):
            sl = pl.ds(i * _LANES, _LANES)
            idx_v[sl] = idx_v[sl] + off
            return carry

        lax.fori_loop(0, idx_per_w // _LANES, add_off, 0)

        def g_desc(c, q):
            idxs = idx_v.at[pl.ds(c * rows_pad, rows)]
            return pltpu.make_async_copy(w_hbm.at[idxs], bufs[q], gsem[q])

        def s_desc(c, q):
            return pltpu.make_async_copy(bufs[q], out_hbm.at[base_c + c],
                                         ssem[q])

        g_desc(0, 0).start()
        g_desc(1, 1).start()

        def group(g, carry):
            c_base = g * _NBUF
            for q in range(_NBUF):
                c = c_base + q
                cn = c + 2
                qn = (q + 2) % _NBUF

                @pl.when(cn < ch_per_w)
                def _():
                    @pl.when(cn >= _NBUF)
                    def _():
                        s_desc(cn - _NBUF, qn).wait()

                    g_desc(cn, qn).start()

                g_desc(c, q).wait()
                s_desc(c, q).start()
            return carry

        lax.fori_loop(0, n_groups, group, 0)
        for q in range(_NBUF):
            s_desc(ch_per_w - _NBUF + q, q).wait()

    return k


def kernel(x, direc, weight):
    b, s = x.shape
    d = weight.shape[1]
    if direc == "LR":
        lo_off = hi_off = 100
    elif direc == "RL":
        lo_off = hi_off = 200
    else:
        lo_off, hi_off = 100, 200
    rows = s // 2
    rows_pad = (rows + 7) // 8 * 8 + 8
    xi = x.astype(jnp.int32).reshape(b, 2, rows)
    xi = jnp.pad(xi, ((0, 0), (0, 0), (0, rows_pad - rows)))
    xi = xi.reshape(b * 2 * rows_pad)
    w3 = weight.reshape(weight.shape[0], d // 128, 128)
    out = _build_sc_gather(b * 2, rows, rows_pad, d, _NW // 2,
                           lo_off, hi_off)(w3, xi)
    return out.reshape(b, s, d)
